# baseline (speedup 1.0000x reference)
import jax
import jax.numpy as jnp
from jax import lax
from jax.experimental import pallas as pl
from jax.experimental.pallas import tpu as pltpu
from jax.experimental.pallas import tpu_sc as plsc

_B = 256
_C = 100
_CP = 112
_EPS = 1e-6
_F32 = jnp.float32
_I32 = jnp.int32
_HI = lax.Precision.HIGHEST
_DN = (((1,), (1,)), ((), ()))

_NW = 16
_RPW = _B // _NW


def _tc_body(t_ref, mu_ref, eta_ref, ep_ref, pred_ref, labr_ref, sim_ref,
             loss_ref):
    T = t_ref[0, 0]
    mu = mu_ref[0, 0]
    eta = eta_ref[0, 0]
    epochi = ep_ref[0, 0]

    x = pred_ref[...] / T
    m = jnp.max(x, axis=1, keepdims=True)
    e = jnp.exp(x - m)
    s = jnp.sum(e, axis=1, keepdims=True)
    p = e / s

    logpe = jnp.log(p + _EPS)
    ent = jnp.sum(p * jnp.log(p + 1e-30), axis=1, keepdims=True)

    K = ent - lax.dot_general(p, logpe, _DN, precision=_HI,
                              preferred_element_type=_F32)
    K = K + _EPS * (1.0 + _C * _EPS)

    ia = lax.broadcasted_iota(_I32, (_B, _B), 0)
    ib = lax.broadcasted_iota(_I32, (_B, _B), 1)
    eyef = (ia == ib).astype(_F32)

    labrf = labr_ref[...].astype(_F32)
    labcf = jnp.sum(eyef * labrf, axis=1, keepdims=True)
    iocf = lax.broadcasted_iota(_I32, (_B, _C), 1).astype(_F32)
    onehot = (labcf == iocf).astype(_F32)

    G = lax.dot_general(p, onehot, _DN, precision=_HI,
                        preferred_element_type=_F32)
    Gt = lax.dot_general(onehot, p, _DN, precision=_HI,
                         preferred_element_type=_F32)
    d_col = jnp.sum(p * onehot, axis=1, keepdims=True)
    d_row = jnp.sum(G * eyef, axis=0, keepdims=True)

    term1 = d_col * jnp.log(d_col / (Gt + _EPS) + _EPS)
    term2 = G * jnp.log(G / (d_row + _EPS) + _EPS)
    S = term1 + term2

    R = lax.dot_general(onehot, sim_ref[...], (((1,), (0,)), ((), ())),
                        precision=_HI, preferred_element_type=_F32)
    simb = lax.dot_general(R, onehot, _DN, precision=_HI,
                           preferred_element_type=_F32)

    triu = (ib > ia).astype(_F32)
    same = (labcf == labrf).astype(_F32)
    same_t = triu * same
    diff_t = triu * (1.0 - same)

    IC_sum = jnp.sum(jnp.abs(K) * same_t)
    simw = jnp.where(epochi == 0, 1.0, simb)
    ISC_sum = jnp.sum(jnp.abs(S * simw) * diff_t)
    same_count = jnp.sum(same_t)
    diff_count = jnp.sum(diff_t)

    IC = jnp.where(same_count != 0.0, IC_sum / same_count, IC_sum)
    ISC = jnp.where(diff_count != 0.0, ISC_sum / diff_count, ISC_sum)
    ISC = jnp.where(ISC != 0.0, 1.0 / (ISC + _EPS) * mu, ISC)
    loss_ref[...] = jnp.broadcast_to(IC * eta + ISC, (1, 1))


def _tc_loss(predicts, labels, sim_all, epoch, T, mu, eta):
    B, C = predicts.shape
    labr = labels.astype(_I32).reshape(1, B)
    t_a = jnp.asarray(T, _F32).reshape(1, 1)
    mu_a = jnp.asarray(mu, _F32).reshape(1, 1)
    eta_a = jnp.asarray(eta, _F32).reshape(1, 1)
    ep_a = jnp.asarray(epoch, _I32).reshape(1, 1)

    loss = pl.pallas_call(
        _tc_body,
        in_specs=[pl.BlockSpec(memory_space=pltpu.SMEM)] * 4 +
                 [pl.BlockSpec(memory_space=pltpu.VMEM)] * 3,
        out_specs=pl.BlockSpec(memory_space=pltpu.VMEM),
        out_shape=jax.ShapeDtypeStruct((1, 1), _F32),
    )(t_a, mu_a, eta_a, ep_a, predicts.astype(_F32), labr,
      sim_all.astype(_F32))
    return loss.reshape(())


def _sc_body(pred_hbm, lab_hbm, zero_hbm, out_hbm,
             scr, labv_ref, col_stage, val_stage,
             sh_col, sh_val, laball, colall, valall, local):
    c = lax.axis_index("c")
    s = lax.axis_index("s")

    @pl.when(c == 0)
    def _worker():
        lanes = lax.iota(_I32, 16)
        base = s * (_RPW * _CP)
        pltpu.sync_copy(pred_hbm.at[pl.ds(base, _RPW * _CP)], scr)
        pltpu.sync_copy(lab_hbm.at[pl.ds(s * _RPW, _RPW)], labv_ref)

        idx1v = jnp.zeros((16,), _I32)
        idx2v = jnp.zeros((16,), _I32)
        neg = jnp.full((16,), -jnp.inf, _F32)
        big16 = jnp.full((16,), 9999, _I32)
        for r in range(_RPW):
            vs = [scr[pl.ds(r * _CP + 16 * j, 16)] for j in range(7)]
            mv = vs[0]
            for v in vs[1:]:
                mv = jnp.maximum(mv, v)
            m1 = jnp.broadcast_to(jnp.max(mv), (16,))
            i1v = big16
            for j, v in enumerate(vs):
                cand = jnp.where(v == m1, lanes + 16 * j, big16)
                i1v = jnp.minimum(i1v, cand)
            i1 = jnp.broadcast_to(jnp.min(i1v), (16,))
            mv2 = neg
            for j, v in enumerate(vs):
                v2 = jnp.where(lanes + 16 * j == i1, neg, v)
                vs[j] = v2
                mv2 = jnp.maximum(mv2, v2)
            m2 = jnp.broadcast_to(jnp.max(mv2), (16,))
            i2v = big16
            for j, v2 in enumerate(vs):
                cand = jnp.where(v2 == m2, lanes + 16 * j, big16)
                i2v = jnp.minimum(i2v, cand)
            i2 = jnp.broadcast_to(jnp.min(i2v), (16,))
            lane_r = (lanes == r)
            idx1v = jnp.where(lane_r, i1, idx1v)
            idx2v = jnp.where(lane_r, i2, idx2v)

        valv = jnp.where(labv_ref[...] == idx1v,
                         jnp.full((16,), 1.0, _F32),
                         jnp.full((16,), 0.0, _F32))
        col_stage[...] = idx2v
        val_stage[...] = valv
        pltpu.sync_copy(col_stage, sh_col.at[pl.ds(s * 16, 16)])
        pltpu.sync_copy(val_stage, sh_val.at[pl.ds(s * 16, 16)])

    plsc.subcore_barrier()

    @pl.when((c == 0) & (s == 0))
    def _tile0():
        lanes = lax.iota(_I32, 16)
        pltpu.sync_copy(lab_hbm, laball)
        pltpu.sync_copy(sh_col, colall)
        pltpu.sync_copy(sh_val, valall)
        pltpu.sync_copy(zero_hbm, local)
        for w in range(_NW):
            roww = laball[pl.ds(w * 16, 16)]
            colw = colall[pl.ds(w * 16, 16)]
            valw = valall[pl.ds(w * 16, 16)]
            flatw = roww * _CP + colw
            for k in range(16):
                plsc.addupdate_scatter(local, [flatw], valw,
                                       mask=(lanes == k))
        pltpu.sync_copy(local, out_hbm)


def _make_sc_simbatch():
    mesh = plsc.VectorSubcoreMesh(core_axis_name="c", subcore_axis_name="s")
    return pl.kernel(
        _sc_body, mesh=mesh,
        compiler_params=pltpu.CompilerParams(needs_layout_passes=False),
        out_type=jax.ShapeDtypeStruct((_CP * _CP,), _F32),
        scratch_types=[
            pltpu.VMEM((_RPW * _CP,), _F32),
            pltpu.VMEM((_RPW,), _I32),
            pltpu.VMEM((16,), _I32),
            pltpu.VMEM((16,), _F32),
            pltpu.VMEM_SHARED((_B,), _I32),
            pltpu.VMEM_SHARED((_B,), _F32),
            pltpu.VMEM((_B,), _I32),
            pltpu.VMEM((_B,), _I32),
            pltpu.VMEM((_B,), _F32),
            pltpu.VMEM((_CP * _CP,), _F32),
        ],
    )


_sc_simbatch_cache = []


def kernel(predicts, labels, sim_all, epoch, T, mu, eta):
    if not _sc_simbatch_cache:
        _sc_simbatch_cache.append(_make_sc_simbatch())
    sc_simbatch = _sc_simbatch_cache[0]
    B, C = predicts.shape
    loss = _tc_loss(predicts, labels, sim_all, epoch, T, mu, eta)
    pred_pad = jnp.pad(predicts.astype(_F32), ((0, 0), (0, _CP - C)),
                       constant_values=-jnp.inf).reshape(B * _CP)
    labi = labels.astype(_I32)
    zeros = jnp.zeros((_CP * _CP,), _F32)
    simb = sc_simbatch(pred_pad, labi, zeros)
    return loss, simb.reshape(_CP, _CP)[:C, :C]


# speedup vs baseline: 1.0761x; 1.0761x over previous
import jax
import jax.numpy as jnp
from jax import lax
from jax.experimental import pallas as pl
from jax.experimental.pallas import tpu as pltpu
from jax.experimental.pallas import tpu_sc as plsc

_B = 256
_C = 100
_CP = 112
_EPS = 1e-6
_F32 = jnp.float32
_I32 = jnp.int32
_HI = lax.Precision.HIGHEST
_DN = (((1,), (1,)), ((), ()))

_NW = 16
_RPW = _B // _NW


def _tc_body(t_ref, mu_ref, eta_ref, ep_ref, pred_ref, labr_ref, sim_ref,
             loss_ref):
    T = t_ref[0, 0]
    mu = mu_ref[0, 0]
    eta = eta_ref[0, 0]
    epochi = ep_ref[0, 0]

    x = pred_ref[...] / T
    m = jnp.max(x, axis=1, keepdims=True)
    e = jnp.exp(x - m)
    s = jnp.sum(e, axis=1, keepdims=True)
    p = e / s

    logpe = jnp.log(p + _EPS)
    ent = jnp.sum(p * jnp.log(p + 1e-30), axis=1, keepdims=True)

    K = ent - lax.dot_general(p, logpe, _DN, precision=_HI,
                              preferred_element_type=_F32)
    K = K + _EPS * (1.0 + _C * _EPS)

    ia = lax.broadcasted_iota(_I32, (_B, _B), 0)
    ib = lax.broadcasted_iota(_I32, (_B, _B), 1)
    eyef = (ia == ib).astype(_F32)

    labrf = labr_ref[...].astype(_F32)
    labcf = jnp.sum(eyef * labrf, axis=1, keepdims=True)
    iocf = lax.broadcasted_iota(_I32, (_B, _C), 1).astype(_F32)
    onehot = (labcf == iocf).astype(_F32)

    G = lax.dot_general(p, onehot, _DN, precision=_HI,
                        preferred_element_type=_F32)
    Gt = lax.dot_general(onehot, p, _DN, precision=_HI,
                         preferred_element_type=_F32)
    d_col = jnp.sum(p * onehot, axis=1, keepdims=True)
    d_row = jnp.sum(G * eyef, axis=0, keepdims=True)

    term1 = d_col * jnp.log(d_col / (Gt + _EPS) + _EPS)
    term2 = G * jnp.log(G / (d_row + _EPS) + _EPS)
    S = term1 + term2

    R = lax.dot_general(onehot, sim_ref[...], (((1,), (0,)), ((), ())),
                        precision=_HI, preferred_element_type=_F32)
    simb = lax.dot_general(R, onehot, _DN, precision=_HI,
                           preferred_element_type=_F32)

    triu = (ib > ia).astype(_F32)
    same = (labcf == labrf).astype(_F32)
    same_t = triu * same
    diff_t = triu * (1.0 - same)

    IC_sum = jnp.sum(jnp.abs(K) * same_t)
    simw = jnp.where(epochi == 0, 1.0, simb)
    ISC_sum = jnp.sum(jnp.abs(S * simw) * diff_t)
    same_count = jnp.sum(same_t)
    diff_count = jnp.sum(diff_t)

    IC = jnp.where(same_count != 0.0, IC_sum / same_count, IC_sum)
    ISC = jnp.where(diff_count != 0.0, ISC_sum / diff_count, ISC_sum)
    ISC = jnp.where(ISC != 0.0, 1.0 / (ISC + _EPS) * mu, ISC)
    loss_ref[...] = jnp.broadcast_to(IC * eta + ISC, (1, 1))


def _tc_loss(predicts, labels, sim_all, epoch, T, mu, eta):
    B, C = predicts.shape
    labr = labels.astype(_I32).reshape(1, B)
    t_a = jnp.asarray(T, _F32).reshape(1, 1)
    mu_a = jnp.asarray(mu, _F32).reshape(1, 1)
    eta_a = jnp.asarray(eta, _F32).reshape(1, 1)
    ep_a = jnp.asarray(epoch, _I32).reshape(1, 1)

    loss = pl.pallas_call(
        _tc_body,
        in_specs=[pl.BlockSpec(memory_space=pltpu.SMEM)] * 4 +
                 [pl.BlockSpec(memory_space=pltpu.VMEM)] * 3,
        out_specs=pl.BlockSpec(memory_space=pltpu.VMEM),
        out_shape=jax.ShapeDtypeStruct((1, 1), _F32),
    )(t_a, mu_a, eta_a, ep_a, predicts.astype(_F32), labr,
      sim_all.astype(_F32))
    return loss.reshape(())


def _sc_body(pred_hbm, lab_hbm, zero_hbm, out_hbm,
             scr, labv_ref, idx_stage, val_stage, sh_acc):
    c = lax.axis_index("c")
    s = lax.axis_index("s")

    @pl.when(c == 0)
    def _worker():
        lanes = lax.iota(_I32, 16)
        base = s * (_RPW * _C)
        pltpu.sync_copy(pred_hbm.at[pl.ds(base, _RPW * _C)],
                        scr.at[pl.ds(0, _RPW * _C)])
        pltpu.sync_copy(lab_hbm.at[pl.ds(s * _RPW, _RPW)], labv_ref)

        idx1v = jnp.zeros((16,), _I32)
        idx2v = jnp.zeros((16,), _I32)
        neg = jnp.full((16,), -jnp.inf, _F32)
        big16 = jnp.full((16,), 9999, _I32)
        for r in range(_RPW):
            vs = []
            for j in range(7):
                iv = lanes + (r * _C + 16 * j)
                v = plsc.load_gather(scr, [iv])
                if j == 6:
                    v = jnp.where(lanes < 4, v, neg)
                vs.append(v)
            mv = vs[0]
            for v in vs[1:]:
                mv = jnp.maximum(mv, v)
            m1 = jnp.broadcast_to(jnp.max(mv), (16,))
            i1v = big16
            for j, v in enumerate(vs):
                cand = jnp.where(v == m1, lanes + 16 * j, big16)
                i1v = jnp.minimum(i1v, cand)
            i1 = jnp.broadcast_to(jnp.min(i1v), (16,))
            mv2 = neg
            for j, v in enumerate(vs):
                v2 = jnp.where(lanes + 16 * j == i1, neg, v)
                vs[j] = v2
                mv2 = jnp.maximum(mv2, v2)
            m2 = jnp.broadcast_to(jnp.max(mv2), (16,))
            i2v = big16
            for j, v2 in enumerate(vs):
                cand = jnp.where(v2 == m2, lanes + 16 * j, big16)
                i2v = jnp.minimum(i2v, cand)
            i2 = jnp.broadcast_to(jnp.min(i2v), (16,))
            lane_r = (lanes == r)
            idx1v = jnp.where(lane_r, i1, idx1v)
            idx2v = jnp.where(lane_r, i2, idx2v)

        valv = jnp.where(labv_ref[...] == idx1v,
                         jnp.full((16,), 1.0, _F32),
                         jnp.full((16,), 0.0, _F32))
        idx_stage[...] = labv_ref[...] * _C + idx2v
        val_stage[...] = valv

    @pl.when((c == 0) & (s == 0))
    def _zero():
        pltpu.sync_copy(zero_hbm, sh_acc)

    plsc.subcore_barrier()

    @pl.when(c == 0)
    def _scatter():
        pltpu.sync_copy(val_stage, sh_acc.at[idx_stage], add=True)

    plsc.subcore_barrier()

    @pl.when((c == 0) & (s == 0))
    def _out():
        pltpu.sync_copy(sh_acc, out_hbm)


def _make_sc_simbatch():
    mesh = plsc.VectorSubcoreMesh(core_axis_name="c", subcore_axis_name="s")
    return pl.kernel(
        _sc_body, mesh=mesh,
        compiler_params=pltpu.CompilerParams(needs_layout_passes=False),
        out_type=jax.ShapeDtypeStruct((_C * _C,), _F32),
        scratch_types=[
            pltpu.VMEM((_RPW * _C + 16,), _F32),
            pltpu.VMEM((_RPW,), _I32),
            pltpu.VMEM((16,), _I32),
            pltpu.VMEM((16,), _F32),
            pltpu.VMEM_SHARED((_C * _C,), _F32),
        ],
    )


_sc_simbatch_cache = []


def kernel(predicts, labels, sim_all, epoch, T, mu, eta):
    if not _sc_simbatch_cache:
        _sc_simbatch_cache.append(_make_sc_simbatch())
    sc_simbatch = _sc_simbatch_cache[0]
    B, C = predicts.shape
    pred_flat = predicts.astype(_F32).reshape(B * C)
    labi = labels.astype(_I32)
    zeros = jnp.zeros((_C * _C,), _F32)
    simb = sc_simbatch(pred_flat, labi, zeros)
    loss = _tc_loss(predicts, labels, sim_all, epoch, T, mu, eta)
    return loss, simb.reshape(C, C)


# speedup vs baseline: 1.1029x; 1.0249x over previous
import jax
import jax.numpy as jnp
from jax import lax
from jax.experimental import pallas as pl
from jax.experimental.pallas import tpu as pltpu
from jax.experimental.pallas import tpu_sc as plsc

_B = 256
_C = 100
_CP = 112
_EPS = 1e-6
_F32 = jnp.float32
_I32 = jnp.int32
_HI = lax.Precision.HIGHEST
_DN = (((1,), (1,)), ((), ()))

_NW = 16
_RPW = _B // _NW


def _tc_body(t_ref, mu_ref, eta_ref, ep_ref, pred_ref, labr_ref, sim_ref,
             loss_ref):
    T = t_ref[0, 0]
    mu = mu_ref[0, 0]
    eta = eta_ref[0, 0]
    epochi = ep_ref[0, 0]

    x = pred_ref[...] / T
    m = jnp.max(x, axis=1, keepdims=True)
    e = jnp.exp(x - m)
    s = jnp.sum(e, axis=1, keepdims=True)
    p = e / s

    logpe = jnp.log(p + _EPS)
    ent = jnp.sum(p * jnp.log(p + 1e-30), axis=1, keepdims=True)

    K = ent - lax.dot_general(p, logpe, _DN, precision=_HI,
                              preferred_element_type=_F32)
    K = K + _EPS * (1.0 + _C * _EPS)

    ia = lax.broadcasted_iota(_I32, (_B, _B), 0)
    ib = lax.broadcasted_iota(_I32, (_B, _B), 1)
    eyef = (ia == ib).astype(_F32)

    labrf = labr_ref[...].astype(_F32)
    labcf = jnp.sum(eyef * labrf, axis=1, keepdims=True)
    iocf = lax.broadcasted_iota(_I32, (_B, _C), 1).astype(_F32)
    onehot = (labcf == iocf).astype(_F32)

    G = lax.dot_general(p, onehot, _DN, precision=_HI,
                        preferred_element_type=_F32)
    Gt = lax.dot_general(onehot, p, _DN, precision=_HI,
                         preferred_element_type=_F32)
    d_col = jnp.sum(p * onehot, axis=1, keepdims=True)
    d_row = jnp.sum(G * eyef, axis=0, keepdims=True)

    term1 = d_col * jnp.log(d_col / (Gt + _EPS) + _EPS)
    term2 = G * jnp.log(G / (d_row + _EPS) + _EPS)
    S = term1 + term2

    R = lax.dot_general(onehot, sim_ref[...], (((1,), (0,)), ((), ())),
                        precision=_HI, preferred_element_type=_F32)
    simb = lax.dot_general(R, onehot, _DN, precision=_HI,
                           preferred_element_type=_F32)

    triu = (ib > ia).astype(_F32)
    same = (labcf == labrf).astype(_F32)
    same_t = triu * same
    diff_t = triu * (1.0 - same)

    IC_sum = jnp.sum(jnp.abs(K) * same_t)
    simw = jnp.where(epochi == 0, 1.0, simb)
    ISC_sum = jnp.sum(jnp.abs(S * simw) * diff_t)
    same_count = jnp.sum(same_t)
    diff_count = jnp.sum(diff_t)

    IC = jnp.where(same_count != 0.0, IC_sum / same_count, IC_sum)
    ISC = jnp.where(diff_count != 0.0, ISC_sum / diff_count, ISC_sum)
    ISC = jnp.where(ISC != 0.0, 1.0 / (ISC + _EPS) * mu, ISC)
    loss_ref[...] = jnp.broadcast_to(IC * eta + ISC, (1, 1))


def _tc_loss(predicts, labels, sim_all, epoch, T, mu, eta):
    B, C = predicts.shape
    labr = labels.astype(_I32).reshape(1, B)
    t_a = jnp.asarray(T, _F32).reshape(1, 1)
    mu_a = jnp.asarray(mu, _F32).reshape(1, 1)
    eta_a = jnp.asarray(eta, _F32).reshape(1, 1)
    ep_a = jnp.asarray(epoch, _I32).reshape(1, 1)

    loss = pl.pallas_call(
        _tc_body,
        in_specs=[pl.BlockSpec(memory_space=pltpu.SMEM)] * 4 +
                 [pl.BlockSpec(memory_space=pltpu.VMEM)] * 3,
        out_specs=pl.BlockSpec(memory_space=pltpu.VMEM),
        out_shape=jax.ShapeDtypeStruct((1, 1), _F32),
    )(t_a, mu_a, eta_a, ep_a, predicts.astype(_F32), labr,
      sim_all.astype(_F32))
    return loss.reshape(())


_CHUNK_BASES = (0, 16, 32, 48, 64, 80, 84)
_ZCH = 640


def _sc_body(pred_hbm, lab_hbm, out_hbm, scr2, labv_ref, idx_stage,
             val_stage, zbuf, obuf, sh_acc):
    c = lax.axis_index("c")
    s = lax.axis_index("s")

    @pl.when(c == 0)
    def _worker():
        lanes = lax.iota(_I32, 16)
        pltpu.sync_copy(pred_hbm.at[pl.ds(s * _RPW, _RPW)], scr2)
        pltpu.sync_copy(lab_hbm.at[pl.ds(s * _RPW, _RPW)], labv_ref)

        zero16 = jnp.zeros((16,), _F32)
        for z in range(_ZCH // 16):
            zbuf[pl.ds(z * 16, 16)] = zero16
        pltpu.sync_copy(zbuf, sh_acc.at[pl.ds(s * _ZCH, _ZCH)])

        idx1v = jnp.zeros((16,), _I32)
        idx2v = jnp.zeros((16,), _I32)
        neg = jnp.full((16,), -jnp.inf, _F32)
        big16 = jnp.full((16,), 9999, _I32)
        for r in range(_RPW):
            vs = [scr2[r, pl.ds(b, 16)] for b in _CHUNK_BASES]
            mv = vs[0]
            for v in vs[1:]:
                mv = jnp.maximum(mv, v)
            m1 = jnp.broadcast_to(jnp.max(mv), (16,))
            i1v = big16
            for b, v in zip(_CHUNK_BASES, vs):
                cand = jnp.where(v == m1, lanes + b, big16)
                i1v = jnp.minimum(i1v, cand)
            i1 = jnp.broadcast_to(jnp.min(i1v), (16,))
            mv2 = neg
            vs2 = []
            for b, v in zip(_CHUNK_BASES, vs):
                v2 = jnp.where(lanes + b == i1, neg, v)
                vs2.append(v2)
                mv2 = jnp.maximum(mv2, v2)
            m2 = jnp.broadcast_to(jnp.max(mv2), (16,))
            i2v = big16
            for b, v2 in zip(_CHUNK_BASES, vs2):
                cand = jnp.where(v2 == m2, lanes + b, big16)
                i2v = jnp.minimum(i2v, cand)
            i2 = jnp.broadcast_to(jnp.min(i2v), (16,))
            lane_r = (lanes == r)
            idx1v = jnp.where(lane_r, i1, idx1v)
            idx2v = jnp.where(lane_r, i2, idx2v)

        valv = jnp.where(labv_ref[...] == idx1v,
                         jnp.full((16,), 1.0, _F32),
                         jnp.full((16,), 0.0, _F32))
        idx_stage[...] = labv_ref[...] * _C + idx2v
        val_stage[...] = valv

    plsc.subcore_barrier()

    @pl.when(c == 0)
    def _scatter():
        pltpu.sync_copy(val_stage, sh_acc.at[idx_stage], add=True)

    plsc.subcore_barrier()

    @pl.when((c == 0) & (s == 0))
    def _out():
        pltpu.sync_copy(sh_acc.at[pl.ds(0, _C * _C)], obuf)
        pltpu.sync_copy(obuf, out_hbm)


def _make_sc_simbatch():
    mesh = plsc.VectorSubcoreMesh(core_axis_name="c", subcore_axis_name="s")
    return pl.kernel(
        _sc_body, mesh=mesh,
        compiler_params=pltpu.CompilerParams(needs_layout_passes=False),
        out_type=jax.ShapeDtypeStruct((_C * _C,), _F32),
        scratch_types=[
            pltpu.VMEM((_RPW, _C), _F32),
            pltpu.VMEM((_RPW,), _I32),
            pltpu.VMEM((16,), _I32),
            pltpu.VMEM((16,), _F32),
            pltpu.VMEM((_ZCH,), _F32),
            pltpu.VMEM((_C * _C,), _F32),
            pltpu.VMEM_SHARED((_NW * _ZCH,), _F32),
        ],
    )


_sc_simbatch_cache = []


def kernel(predicts, labels, sim_all, epoch, T, mu, eta):
    if not _sc_simbatch_cache:
        _sc_simbatch_cache.append(_make_sc_simbatch())
    sc_simbatch = _sc_simbatch_cache[0]
    B, C = predicts.shape
    labi = labels.astype(_I32)
    simb = sc_simbatch(predicts.astype(_F32), labi)
    loss = _tc_loss(predicts, labels, sim_all, epoch, T, mu, eta)
    return loss, simb.reshape(C, C)


# speedup vs baseline: 1.1684x; 1.0594x over previous
import jax
import jax.numpy as jnp
from jax import lax
from jax.experimental import pallas as pl
from jax.experimental.pallas import tpu as pltpu
from jax.experimental.pallas import tpu_sc as plsc

_B = 256
_C = 100
_CP = 112
_EPS = 1e-6
_F32 = jnp.float32
_I32 = jnp.int32
_HI = lax.Precision.HIGHEST
_DN = (((1,), (1,)), ((), ()))

_NW = 16
_RPW = _B // _NW


def _tc_body(t_ref, mu_ref, eta_ref, ep_ref, pred_ref, labr_ref, sim_ref,
             loss_ref):
    T = t_ref[0, 0]
    mu = mu_ref[0, 0]
    eta = eta_ref[0, 0]
    epochi = ep_ref[0, 0]

    x = pred_ref[...] / T
    m = jnp.max(x, axis=1, keepdims=True)
    e = jnp.exp(x - m)
    s = jnp.sum(e, axis=1, keepdims=True)
    p = e / s

    logpe = jnp.log(p + _EPS)
    ent = jnp.sum(p * jnp.log(p + 1e-30), axis=1, keepdims=True)

    K = ent - lax.dot_general(p, logpe, _DN, precision=_HI,
                              preferred_element_type=_F32)
    K = K + _EPS * (1.0 + _C * _EPS)

    ia = lax.broadcasted_iota(_I32, (_B, _B), 0)
    ib = lax.broadcasted_iota(_I32, (_B, _B), 1)
    eyef = (ia == ib).astype(_F32)

    labrf = labr_ref[...].astype(_F32)
    labcf = jnp.sum(eyef * labrf, axis=1, keepdims=True)
    iocf = lax.broadcasted_iota(_I32, (_B, _C), 1).astype(_F32)
    onehot = (labcf == iocf).astype(_F32)

    G = lax.dot_general(p, onehot, _DN, precision=_HI,
                        preferred_element_type=_F32)
    Gt = lax.dot_general(onehot, p, _DN, precision=_HI,
                         preferred_element_type=_F32)
    d_col = jnp.sum(p * onehot, axis=1, keepdims=True)
    d_row = jnp.sum(G * eyef, axis=0, keepdims=True)

    term1 = d_col * jnp.log(d_col / (Gt + _EPS) + _EPS)
    term2 = G * jnp.log(G / (d_row + _EPS) + _EPS)
    S = term1 + term2

    R = lax.dot_general(onehot, sim_ref[...], (((1,), (0,)), ((), ())),
                        precision=_HI, preferred_element_type=_F32)
    simb = lax.dot_general(R, onehot, _DN, precision=_HI,
                           preferred_element_type=_F32)

    triu = (ib > ia).astype(_F32)
    same = (labcf == labrf).astype(_F32)
    same_t = triu * same
    diff_t = triu * (1.0 - same)

    IC_sum = jnp.sum(jnp.abs(K) * same_t)
    simw = jnp.where(epochi == 0, 1.0, simb)
    ISC_sum = jnp.sum(jnp.abs(S * simw) * diff_t)
    same_count = jnp.sum(same_t)
    diff_count = jnp.sum(diff_t)

    IC = jnp.where(same_count != 0.0, IC_sum / same_count, IC_sum)
    ISC = jnp.where(diff_count != 0.0, ISC_sum / diff_count, ISC_sum)
    ISC = jnp.where(ISC != 0.0, 1.0 / (ISC + _EPS) * mu, ISC)
    loss_ref[...] = jnp.broadcast_to(IC * eta + ISC, (1, 1))


def _tc_loss(predicts, labels, sim_all, epoch, T, mu, eta):
    B, C = predicts.shape
    labr = labels.astype(_I32).reshape(1, B)
    t_a = jnp.asarray(T, _F32).reshape(1, 1)
    mu_a = jnp.asarray(mu, _F32).reshape(1, 1)
    eta_a = jnp.asarray(eta, _F32).reshape(1, 1)
    ep_a = jnp.asarray(epoch, _I32).reshape(1, 1)

    loss = pl.pallas_call(
        _tc_body,
        in_specs=[pl.BlockSpec(memory_space=pltpu.SMEM)] * 4 +
                 [pl.BlockSpec(memory_space=pltpu.VMEM)] * 3,
        out_specs=pl.BlockSpec(memory_space=pltpu.VMEM),
        out_shape=jax.ShapeDtypeStruct((1, 1), _F32),
    )(t_a, mu_a, eta_a, ep_a, predicts.astype(_F32), labr,
      sim_all.astype(_F32))
    return loss.reshape(())


_CHUNK_BASES = (0, 16, 32, 48, 64, 80, 84)
_ZCH = 640


def _sc_body(pred_hbm, lab_hbm, out_hbm, scr2, labv_ref, idx_stage,
             val_stage, zbuf, obuf, sh_acc):
    c = lax.axis_index("c")
    s = lax.axis_index("s")

    @pl.when(c == 0)
    def _worker():
        lanes = lax.iota(_I32, 16)
        pltpu.sync_copy(pred_hbm.at[pl.ds(s * _RPW, _RPW)], scr2)
        pltpu.sync_copy(lab_hbm.at[pl.ds(s * _RPW, _RPW)], labv_ref)

        zero16 = jnp.zeros((16,), _F32)
        for z in range(_ZCH // 16):
            zbuf[pl.ds(z * 16, 16)] = zero16
        pltpu.sync_copy(zbuf, sh_acc.at[pl.ds(s * _ZCH, _ZCH)])

        neg = jnp.full((16,), -jnp.inf, _F32)
        big16 = jnp.full((16,), 9999, _I32)

        def _row(r, carry):
            idx1v, idx2v = carry
            vs = [scr2[r, pl.ds(b, 16)] for b in _CHUNK_BASES]
            mv = vs[0]
            for v in vs[1:]:
                mv = jnp.maximum(mv, v)
            m1 = jnp.broadcast_to(jnp.max(mv), (16,))
            i1v = big16
            for b, v in zip(_CHUNK_BASES, vs):
                cand = jnp.where(v == m1, lanes + b, big16)
                i1v = jnp.minimum(i1v, cand)
            i1 = jnp.broadcast_to(jnp.min(i1v), (16,))
            mv2 = neg
            vs2 = []
            for b, v in zip(_CHUNK_BASES, vs):
                v2 = jnp.where(lanes + b == i1, neg, v)
                vs2.append(v2)
                mv2 = jnp.maximum(mv2, v2)
            m2 = jnp.broadcast_to(jnp.max(mv2), (16,))
            i2v = big16
            for b, v2 in zip(_CHUNK_BASES, vs2):
                cand = jnp.where(v2 == m2, lanes + b, big16)
                i2v = jnp.minimum(i2v, cand)
            i2 = jnp.broadcast_to(jnp.min(i2v), (16,))
            lane_r = (lanes == r)
            idx1v = jnp.where(lane_r, i1, idx1v)
            idx2v = jnp.where(lane_r, i2, idx2v)
            return idx1v, idx2v

        idx1v, idx2v = lax.fori_loop(
            0, _RPW, _row,
            (jnp.zeros((16,), _I32), jnp.zeros((16,), _I32)))

        valv = jnp.where(labv_ref[...] == idx1v,
                         jnp.full((16,), 1.0, _F32),
                         jnp.full((16,), 0.0, _F32))
        idx_stage[...] = labv_ref[...] * _C + idx2v
        val_stage[...] = valv

    plsc.subcore_barrier()

    @pl.when(c == 0)
    def _scatter():
        pltpu.sync_copy(val_stage, sh_acc.at[idx_stage], add=True)

    plsc.subcore_barrier()

    @pl.when((c == 0) & (s == 0))
    def _out():
        pltpu.sync_copy(sh_acc.at[pl.ds(0, _C * _C)], obuf)
        pltpu.sync_copy(obuf, out_hbm)


def _make_sc_simbatch():
    mesh = plsc.VectorSubcoreMesh(core_axis_name="c", subcore_axis_name="s")
    return pl.kernel(
        _sc_body, mesh=mesh,
        compiler_params=pltpu.CompilerParams(needs_layout_passes=False),
        out_type=jax.ShapeDtypeStruct((_C * _C,), _F32),
        scratch_types=[
            pltpu.VMEM((_RPW, _C), _F32),
            pltpu.VMEM((_RPW,), _I32),
            pltpu.VMEM((16,), _I32),
            pltpu.VMEM((16,), _F32),
            pltpu.VMEM((_ZCH,), _F32),
            pltpu.VMEM((_C * _C,), _F32),
            pltpu.VMEM_SHARED((_NW * _ZCH,), _F32),
        ],
    )


_sc_simbatch_cache = []


def kernel(predicts, labels, sim_all, epoch, T, mu, eta):
    if not _sc_simbatch_cache:
        _sc_simbatch_cache.append(_make_sc_simbatch())
    sc_simbatch = _sc_simbatch_cache[0]
    B, C = predicts.shape
    labi = labels.astype(_I32)
    simb = sc_simbatch(predicts.astype(_F32), labi)
    loss = _tc_loss(predicts, labels, sim_all, epoch, T, mu, eta)
    return loss, simb.reshape(C, C)
